# Initial kernel scaffold; baseline (speedup 1.0000x reference)
import jax
import jax.numpy as jnp
from jax.experimental import pallas as pl


def kernel(x, edge_index, batch, W_in_l, W_in_r, b_in, W_h_l, W_h_r, b_h, W_out_l, W_out_r, b_out):
    raise NotImplementedError("write your pallas kernel here")


# speedup vs baseline: 10.3431x; 10.3431x over previous
import functools

import jax
import jax.numpy as jnp
from jax import lax
from jax.experimental import pallas as pl
from jax.experimental.pallas import tpu as pltpu
from jax.experimental.pallas import tpu_sc as plsc

N = 10000
E = 320000
BATCHES = 8
N_PAD = 10240
C_REAL = 5000
C_PAD = 5120
E_PAD = 327680
P_IDX = 10100
K = 128
NC = 2
NS = 16
RB = 1024


def _make_sc_seg_sum(n_rows, width, shift):
    rows_per_tile = n_rows // NS
    chunks_per_tile = E_PAD // (NC * NS * K)
    ZR = 64
    mesh = plsc.VectorSubcoreMesh(core_axis_name="c", subcore_axis_name="s")

    @functools.partial(
        pl.kernel,
        out_type=jax.ShapeDtypeStruct((NC * n_rows, width), jnp.float32),
        mesh=mesh,
        scratch_types=[
            pltpu.VMEM((K,), jnp.int32),
            pltpu.VMEM((K,), jnp.int32),
            pltpu.VMEM((K, width), jnp.float32),
            pltpu.VMEM((ZR, width), jnp.float32),
            pltpu.VMEM_SHARED((n_rows, width), jnp.float32),
            pltpu.SemaphoreType.DMA,
        ],
        compiler_params=pltpu.CompilerParams(use_tc_tiling_on_sc=False),
    )
    def seg_sum(tab_hbm, src_hbm, dst_hbm, out_hbm, src_v, dst_v, rows_v,
                zb_v, acc_sh, sem):
        c = lax.axis_index("c")
        s = lax.axis_index("s")
        wid = s * NC + c
        zero16 = jnp.zeros((16,), jnp.float32)
        for r in range(ZR):
            for j in range(width // 16):
                zb_v[r, pl.ds(j * 16, 16)] = zero16
        r0 = s * rows_per_tile
        for t in range(rows_per_tile // ZR):
            pltpu.sync_copy(zb_v, acc_sh.at[pl.ds(r0 + t * ZR, ZR)])
        plsc.subcore_barrier()

        base0 = wid * chunks_per_tile * K

        def body(i, carry):
            base = pl.multiple_of(base0 + i * K, K)
            pltpu.sync_copy(src_hbm.at[pl.ds(base, K)], src_v)
            pltpu.sync_copy(dst_hbm.at[pl.ds(base, K)], dst_v)
            if shift:
                for j in range(K // 16):
                    sl = pl.ds(j * 16, 16)
                    src_v[sl] = lax.shift_right_logical(src_v[sl], 1)
                    dst_v[sl] = lax.shift_right_logical(dst_v[sl], 1)
            pltpu.async_copy(tab_hbm.at[src_v], rows_v, sem).wait()
            pltpu.sync_copy(rows_v, acc_sh.at[dst_v], add=True)
            return carry

        lax.fori_loop(0, chunks_per_tile, body, 0)
        plsc.subcore_barrier()
        pltpu.sync_copy(
            acc_sh.at[pl.ds(r0, rows_per_tile)],
            out_hbm.at[pl.ds(c * n_rows + r0, rows_per_tile)])

    return seg_sum


@functools.cache
def _get_sc_seg_sum(n_rows, width, shift):
    return _make_sc_seg_sum(n_rows, width, shift)


def _seg_sum_sc(table, src, dst, n_rows, width, shift):
    return _get_sc_seg_sum(n_rows, width, shift)(table, src, dst)


def _p1_body(x_ref, wl_ref, wr_ref, y1e_ref, z1_ref):
    i = pl.program_id(0)
    xb = x_ref[...]
    y = jnp.dot(xb, wl_ref[...], preferred_element_type=jnp.float32)
    rows = i * RB + lax.broadcasted_iota(jnp.int32, (RB, 1), 0)
    rmask = jnp.where(rows < N, 1.0, 0.0)
    col16 = lax.broadcasted_iota(jnp.int32, (1, 16), 1)
    extra = jnp.where(col16 == 0, rmask, 0.0)
    y1e_ref[...] = jnp.concatenate([y, extra], axis=1)
    z1_ref[...] = jnp.dot(xb, wr_ref[...], preferred_element_type=jnp.float32)


def _p1(x_pad, W_in_l, W_in_r):
    return pl.pallas_call(
        _p1_body,
        grid=(N_PAD // RB,),
        in_specs=[
            pl.BlockSpec((RB, 128), lambda i: (i, 0)),
            pl.BlockSpec((128, 64), lambda i: (0, 0)),
            pl.BlockSpec((128, 64), lambda i: (0, 0)),
        ],
        out_specs=[
            pl.BlockSpec((RB, 80), lambda i: (i, 0)),
            pl.BlockSpec((RB, 64), lambda i: (i, 0)),
        ],
        out_shape=[
            jax.ShapeDtypeStruct((N_PAD, 80), jnp.float32),
            jax.ShapeDtypeStruct((N_PAD, 64), jnp.float32),
        ],
    )(x_pad, W_in_l, W_in_r)


def _p2_body(sa_ref, sb_ref, z1_ref, bi_ref, whl_ref, whr_ref, y2_ref, z2_ref):
    i = pl.program_id(0)
    sblk = sa_ref[...] + sb_ref[...]
    cnt = jnp.maximum(sblk[:, 64:65], 1.0)
    h = sblk[:, :64] / cnt + z1_ref[...] + bi_ref[...]
    h = jnp.maximum(h, 0.0)
    nrm = jnp.maximum(jnp.sqrt(jnp.sum(h * h, axis=1, keepdims=True)), 1e-12)
    h = h / nrm
    rows = i * RB + lax.broadcasted_iota(jnp.int32, (RB, 1), 0)
    h = jnp.where(rows < N, h, 0.0)
    y2_ref[...] = jnp.dot(h, whl_ref[...], preferred_element_type=jnp.float32)
    z2_ref[...] = jnp.dot(h, whr_ref[...], preferred_element_type=jnp.float32)


def _p2(s1, z1, bi, W_h_l, W_h_r):
    nb = N_PAD // RB
    return pl.pallas_call(
        _p2_body,
        grid=(nb,),
        in_specs=[
            pl.BlockSpec((RB, 80), lambda i: (i, 0)),
            pl.BlockSpec((RB, 80), lambda i, nb=nb: (nb + i, 0)),
            pl.BlockSpec((RB, 64), lambda i: (i, 0)),
            pl.BlockSpec((1, 64), lambda i: (0, 0)),
            pl.BlockSpec((64, 64), lambda i: (0, 0)),
            pl.BlockSpec((64, 64), lambda i: (0, 0)),
        ],
        out_specs=[
            pl.BlockSpec((RB, 64), lambda i: (i, 0)),
            pl.BlockSpec((RB, 64), lambda i: (i, 0)),
        ],
        out_shape=[
            jax.ShapeDtypeStruct((N_PAD, 64), jnp.float32),
            jax.ShapeDtypeStruct((N_PAD, 64), jnp.float32),
        ],
    )(s1, s1, z1, bi, W_h_l, W_h_r)


def _p3_body(sa_ref, sb_ref, ca_ref, cb_ref, z2_ref, bh_ref, wol_ref, wor_ref,
             bo_ref, y3_ref, z3_ref):
    i = pl.program_id(0)
    cnt = jnp.maximum(ca_ref[:, 64:65] + cb_ref[:, 64:65], 1.0)
    sblk = sa_ref[...] + sb_ref[...]
    h = jnp.maximum(sblk / cnt + z2_ref[...] + bh_ref[...], 0.0)
    nrm = jnp.maximum(jnp.sqrt(jnp.sum(h * h, axis=1, keepdims=True)), 1e-12)
    h = h / nrm
    rows = i * RB + lax.broadcasted_iota(jnp.int32, (RB, 1), 0)
    h = jnp.where(rows < N, h, 0.0)
    rj = lax.broadcasted_iota(jnp.int32, (RB // 2, RB), 0)
    ci = lax.broadcasted_iota(jnp.int32, (RB // 2, RB), 1)
    pair = jnp.where(lax.shift_right_logical(ci, 1) == rj, 0.5, 0.0)
    x2 = jnp.dot(pair, h, preferred_element_type=jnp.float32)
    gc = i * (RB // 2) + lax.broadcasted_iota(jnp.int32, (RB // 2, 1), 0)
    cmask = gc < C_REAL
    col16 = lax.broadcasted_iota(jnp.int32, (1, 16), 1)
    y3 = jnp.dot(x2, wol_ref[...], preferred_element_type=jnp.float32)
    y3 = y3 + jnp.where(col16 == 10, 1.0, 0.0)
    y3_ref[...] = jnp.where(cmask, y3, 0.0)
    z3_ref[...] = (jnp.dot(x2, wor_ref[...], preferred_element_type=jnp.float32)
                   + bo_ref[...])


def _p3(s2, s1, z2, bh, Wl3, Wr3, bo):
    nb = N_PAD // RB
    return pl.pallas_call(
        _p3_body,
        grid=(nb,),
        in_specs=[
            pl.BlockSpec((RB, 64), lambda i: (i, 0)),
            pl.BlockSpec((RB, 64), lambda i, nb=nb: (nb + i, 0)),
            pl.BlockSpec((RB, 80), lambda i: (i, 0)),
            pl.BlockSpec((RB, 80), lambda i, nb=nb: (nb + i, 0)),
            pl.BlockSpec((RB, 64), lambda i: (i, 0)),
            pl.BlockSpec((1, 64), lambda i: (0, 0)),
            pl.BlockSpec((64, 16), lambda i: (0, 0)),
            pl.BlockSpec((64, 16), lambda i: (0, 0)),
            pl.BlockSpec((1, 16), lambda i: (0, 0)),
        ],
        out_specs=[
            pl.BlockSpec((RB // 2, 16), lambda i: (i, 0)),
            pl.BlockSpec((RB // 2, 16), lambda i: (i, 0)),
        ],
        out_shape=[
            jax.ShapeDtypeStruct((C_PAD, 16), jnp.float32),
            jax.ShapeDtypeStruct((C_PAD, 16), jnp.float32),
        ],
    )(s2, s2, s1, s1, z2, bh, Wl3, Wr3, bo)


def _p4_body(sa_ref, sb_ref, z3_ref, b2_ref, out_ref):
    sblk = sa_ref[...] + sb_ref[...]
    cnt2 = jnp.maximum(sblk[:, 10:11], 1.0)
    o = sblk / cnt2 + z3_ref[...]
    col16 = lax.broadcasted_iota(jnp.int32, (1, 16), 1)
    cm = col16 < 10
    o = jnp.where(cm, o, 0.0)
    nrm = jnp.maximum(jnp.sqrt(jnp.sum(o * o, axis=1, keepdims=True)), 1e-12)
    o = o / nrm
    b2 = b2_ref[...]
    rows = []
    for b in range(BATCHES):
        m = jnp.where(b2 == b, 1.0, 0.0)
        gs = jnp.sum(o * m, axis=0, keepdims=True)
        gcnt = jnp.maximum(jnp.sum(m), 1.0)
        rows.append(gs / gcnt)
    out = jnp.concatenate(rows, axis=0)
    neg = jnp.where(cm, out, -1e30)
    mx = jnp.max(neg, axis=1, keepdims=True)
    e = jnp.where(cm, jnp.exp(out - mx), 0.0)
    lse = jnp.log(jnp.sum(e, axis=1, keepdims=True))
    out_ref[...] = out - mx - lse


def _p4(s3, z3, batch2):
    return pl.pallas_call(
        _p4_body,
        grid=(1,),
        in_specs=[
            pl.BlockSpec((C_PAD, 16), lambda i: (0, 0)),
            pl.BlockSpec((C_PAD, 16), lambda i: (1, 0)),
            pl.BlockSpec((C_PAD, 16), lambda i: (0, 0)),
            pl.BlockSpec((C_PAD, 1), lambda i: (0, 0)),
        ],
        out_specs=pl.BlockSpec((BATCHES, 16), lambda i: (0, 0)),
        out_shape=jax.ShapeDtypeStruct((BATCHES, 16), jnp.float32),
    )(s3, s3, z3, batch2)


def kernel(x, edge_index, batch, W_in_l, W_in_r, b_in, W_h_l, W_h_r, b_h,
           W_out_l, W_out_r, b_out):
    pad = jnp.full((E_PAD - E,), P_IDX, jnp.int32)
    src = jnp.concatenate([edge_index[0], pad])
    dst = jnp.concatenate([edge_index[1], pad])
    x_pad = jnp.pad(x, ((0, N_PAD - N), (0, 0)))
    batch2 = jnp.concatenate(
        [batch[0::2], jnp.full((C_PAD - C_REAL,), BATCHES, jnp.int32)]
    ).reshape(C_PAD, 1)
    Wl3 = jnp.pad(W_out_l, ((0, 0), (0, 6)))
    Wr3 = jnp.pad(W_out_r, ((0, 0), (0, 6)))
    bo = jnp.pad(b_out, (0, 6)).reshape(1, 16)
    bi = b_in.reshape(1, 64)
    bh = b_h.reshape(1, 64)

    y1e, z1 = _p1(x_pad, W_in_l, W_in_r)
    s1 = _seg_sum_sc(y1e, src, dst, N_PAD, 80, False)
    y2, z2 = _p2(s1, z1, bi, W_h_l, W_h_r)
    s2 = _seg_sum_sc(y2, src, dst, N_PAD, 64, False)
    y3, z3 = _p3(s2, s1, z2, bh, Wl3, Wr3, bo)
    s3 = _seg_sum_sc(y3, src, dst, C_PAD, 16, True)
    out = _p4(s3, z3, batch2)
    return out[:, :10]


# speedup vs baseline: 13.8668x; 1.3407x over previous
import functools

import jax
import jax.numpy as jnp
from jax import lax
from jax.experimental import pallas as pl
from jax.experimental.pallas import tpu as pltpu
from jax.experimental.pallas import tpu_sc as plsc

N = 10000
E = 320000
BATCHES = 8
N_PAD = 10240
C_REAL = 5000
C_PAD = 5120
E_PAD = 327680
P_IDX = 10100
K = 128
NC = 2
NS = 16
RB = 1024


def _make_sc_seg_sum(n_rows, width, shift):
    rows_per_tile = n_rows // NS
    chunks_per_tile = E_PAD // (NC * NS * K)
    NB = 4
    groups = chunks_per_tile // NB
    ZR = 64
    mesh = plsc.VectorSubcoreMesh(core_axis_name="c", subcore_axis_name="s")

    @functools.partial(
        pl.kernel,
        out_type=jax.ShapeDtypeStruct((NC * n_rows, width), jnp.float32),
        mesh=mesh,
        scratch_types=[
            pltpu.VMEM((NB, K), jnp.int32),
            pltpu.VMEM((NB, K), jnp.int32),
            pltpu.VMEM((NB, K, width), jnp.float32),
            pltpu.VMEM((ZR, width), jnp.float32),
            pltpu.VMEM_SHARED((n_rows, width), jnp.float32),
            pltpu.SemaphoreType.DMA,
            pltpu.SemaphoreType.DMA,
        ],
        compiler_params=pltpu.CompilerParams(use_tc_tiling_on_sc=False),
    )
    def seg_sum(tab_hbm, src_hbm, dst_hbm, out_hbm, src_v, dst_v, rows_v,
                zb_v, acc_sh, gsem, ssem):
        c = lax.axis_index("c")
        s = lax.axis_index("s")
        wid = s * NC + c
        zero16 = jnp.zeros((16,), jnp.float32)
        for r in range(ZR):
            for j in range(width // 16):
                zb_v[r, pl.ds(j * 16, 16)] = zero16
        r0 = s * rows_per_tile
        for t in range(rows_per_tile // ZR):
            pltpu.sync_copy(zb_v, acc_sh.at[pl.ds(r0 + t * ZR, ZR)])
        plsc.subcore_barrier()

        def body(t, carry):
            row0 = wid * chunks_per_tile + t * NB
            pltpu.sync_copy(src_hbm.at[pl.ds(row0, NB)], src_v)
            pltpu.sync_copy(dst_hbm.at[pl.ds(row0, NB)], dst_v)
            if shift:
                for b in range(NB):
                    for j in range(K // 16):
                        sl = pl.ds(j * 16, 16)
                        src_v[b, sl] = lax.shift_right_logical(src_v[b, sl], 1)
                        dst_v[b, sl] = lax.shift_right_logical(dst_v[b, sl], 1)
            gds = [pltpu.async_copy(tab_hbm.at[src_v.at[b]], rows_v.at[b], gsem)
                   for b in range(NB)]
            sds = []
            for b in range(NB):
                gds[b].wait()
                sds.append(pltpu.async_copy(rows_v.at[b], acc_sh.at[dst_v.at[b]],
                                            ssem, add=True))
            for b in range(NB):
                sds[b].wait()
            return carry

        lax.fori_loop(0, groups, body, 0)
        plsc.subcore_barrier()
        pltpu.sync_copy(
            acc_sh.at[pl.ds(r0, rows_per_tile)],
            out_hbm.at[pl.ds(c * n_rows + r0, rows_per_tile)])

    return seg_sum


@functools.cache
def _get_sc_seg_sum(n_rows, width, shift):
    return _make_sc_seg_sum(n_rows, width, shift)


def _seg_sum_sc(table, src, dst, n_rows, width, shift):
    return _get_sc_seg_sum(n_rows, width, shift)(table, src, dst)


def _p1_body(x_ref, wl_ref, wr_ref, y1e_ref, z1_ref):
    i = pl.program_id(0)
    xb = x_ref[...]
    y = jnp.dot(xb, wl_ref[...], preferred_element_type=jnp.float32)
    rows = i * RB + lax.broadcasted_iota(jnp.int32, (RB, 1), 0)
    rmask = jnp.where(rows < N, 1.0, 0.0)
    col16 = lax.broadcasted_iota(jnp.int32, (1, 16), 1)
    extra = jnp.where(col16 == 0, rmask, 0.0)
    y1e_ref[...] = jnp.concatenate([y, extra], axis=1)
    z1_ref[...] = jnp.dot(xb, wr_ref[...], preferred_element_type=jnp.float32)


def _p1(x_pad, W_in_l, W_in_r):
    return pl.pallas_call(
        _p1_body,
        grid=(N_PAD // RB,),
        in_specs=[
            pl.BlockSpec((RB, 128), lambda i: (i, 0)),
            pl.BlockSpec((128, 64), lambda i: (0, 0)),
            pl.BlockSpec((128, 64), lambda i: (0, 0)),
        ],
        out_specs=[
            pl.BlockSpec((RB, 80), lambda i: (i, 0)),
            pl.BlockSpec((RB, 64), lambda i: (i, 0)),
        ],
        out_shape=[
            jax.ShapeDtypeStruct((N_PAD, 80), jnp.float32),
            jax.ShapeDtypeStruct((N_PAD, 64), jnp.float32),
        ],
    )(x_pad, W_in_l, W_in_r)


def _p2_body(sa_ref, sb_ref, z1_ref, bi_ref, whl_ref, whr_ref, y2_ref, z2_ref):
    i = pl.program_id(0)
    sblk = sa_ref[...] + sb_ref[...]
    cnt = jnp.maximum(sblk[:, 64:65], 1.0)
    h = sblk[:, :64] / cnt + z1_ref[...] + bi_ref[...]
    h = jnp.maximum(h, 0.0)
    nrm = jnp.maximum(jnp.sqrt(jnp.sum(h * h, axis=1, keepdims=True)), 1e-12)
    h = h / nrm
    rows = i * RB + lax.broadcasted_iota(jnp.int32, (RB, 1), 0)
    h = jnp.where(rows < N, h, 0.0)
    y2_ref[...] = jnp.dot(h, whl_ref[...], preferred_element_type=jnp.float32)
    z2_ref[...] = jnp.dot(h, whr_ref[...], preferred_element_type=jnp.float32)


def _p2(s1, z1, bi, W_h_l, W_h_r):
    nb = N_PAD // RB
    return pl.pallas_call(
        _p2_body,
        grid=(nb,),
        in_specs=[
            pl.BlockSpec((RB, 80), lambda i: (i, 0)),
            pl.BlockSpec((RB, 80), lambda i, nb=nb: (nb + i, 0)),
            pl.BlockSpec((RB, 64), lambda i: (i, 0)),
            pl.BlockSpec((1, 64), lambda i: (0, 0)),
            pl.BlockSpec((64, 64), lambda i: (0, 0)),
            pl.BlockSpec((64, 64), lambda i: (0, 0)),
        ],
        out_specs=[
            pl.BlockSpec((RB, 64), lambda i: (i, 0)),
            pl.BlockSpec((RB, 64), lambda i: (i, 0)),
        ],
        out_shape=[
            jax.ShapeDtypeStruct((N_PAD, 64), jnp.float32),
            jax.ShapeDtypeStruct((N_PAD, 64), jnp.float32),
        ],
    )(s1, s1, z1, bi, W_h_l, W_h_r)


def _p3_body(sa_ref, sb_ref, ca_ref, cb_ref, z2_ref, bh_ref, wol_ref, wor_ref,
             bo_ref, y3_ref, z3_ref):
    i = pl.program_id(0)
    cnt = jnp.maximum(ca_ref[:, 64:65] + cb_ref[:, 64:65], 1.0)
    sblk = sa_ref[...] + sb_ref[...]
    h = jnp.maximum(sblk / cnt + z2_ref[...] + bh_ref[...], 0.0)
    nrm = jnp.maximum(jnp.sqrt(jnp.sum(h * h, axis=1, keepdims=True)), 1e-12)
    h = h / nrm
    rows = i * RB + lax.broadcasted_iota(jnp.int32, (RB, 1), 0)
    h = jnp.where(rows < N, h, 0.0)
    rj = lax.broadcasted_iota(jnp.int32, (RB // 2, RB), 0)
    ci = lax.broadcasted_iota(jnp.int32, (RB // 2, RB), 1)
    pair = jnp.where(lax.shift_right_logical(ci, 1) == rj, 0.5, 0.0)
    x2 = jnp.dot(pair, h, preferred_element_type=jnp.float32)
    gc = i * (RB // 2) + lax.broadcasted_iota(jnp.int32, (RB // 2, 1), 0)
    cmask = gc < C_REAL
    col16 = lax.broadcasted_iota(jnp.int32, (1, 16), 1)
    y3 = jnp.dot(x2, wol_ref[...], preferred_element_type=jnp.float32)
    y3 = y3 + jnp.where(col16 == 10, 1.0, 0.0)
    y3_ref[...] = jnp.where(cmask, y3, 0.0)
    z3_ref[...] = (jnp.dot(x2, wor_ref[...], preferred_element_type=jnp.float32)
                   + bo_ref[...])


def _p3(s2, s1, z2, bh, Wl3, Wr3, bo):
    nb = N_PAD // RB
    return pl.pallas_call(
        _p3_body,
        grid=(nb,),
        in_specs=[
            pl.BlockSpec((RB, 64), lambda i: (i, 0)),
            pl.BlockSpec((RB, 64), lambda i, nb=nb: (nb + i, 0)),
            pl.BlockSpec((RB, 80), lambda i: (i, 0)),
            pl.BlockSpec((RB, 80), lambda i, nb=nb: (nb + i, 0)),
            pl.BlockSpec((RB, 64), lambda i: (i, 0)),
            pl.BlockSpec((1, 64), lambda i: (0, 0)),
            pl.BlockSpec((64, 16), lambda i: (0, 0)),
            pl.BlockSpec((64, 16), lambda i: (0, 0)),
            pl.BlockSpec((1, 16), lambda i: (0, 0)),
        ],
        out_specs=[
            pl.BlockSpec((RB // 2, 16), lambda i: (i, 0)),
            pl.BlockSpec((RB // 2, 16), lambda i: (i, 0)),
        ],
        out_shape=[
            jax.ShapeDtypeStruct((C_PAD, 16), jnp.float32),
            jax.ShapeDtypeStruct((C_PAD, 16), jnp.float32),
        ],
    )(s2, s2, s1, s1, z2, bh, Wl3, Wr3, bo)


def _p4_body(sa_ref, sb_ref, z3_ref, b2_ref, out_ref):
    sblk = sa_ref[...] + sb_ref[...]
    cnt2 = jnp.maximum(sblk[:, 10:11], 1.0)
    o = sblk / cnt2 + z3_ref[...]
    col16 = lax.broadcasted_iota(jnp.int32, (1, 16), 1)
    cm = col16 < 10
    o = jnp.where(cm, o, 0.0)
    nrm = jnp.maximum(jnp.sqrt(jnp.sum(o * o, axis=1, keepdims=True)), 1e-12)
    o = o / nrm
    b2 = b2_ref[...]
    col8 = lax.broadcasted_iota(jnp.int32, (1, BATCHES), 1)
    onehot = jnp.where(b2 == col8, 1.0, 0.0)
    dn = (((0,), (0,)), ((), ()))
    gs = lax.dot_general(onehot, o, dn, preferred_element_type=jnp.float32)
    gc = lax.dot_general(onehot, jnp.ones_like(o), dn,
                         preferred_element_type=jnp.float32)
    out = gs / jnp.maximum(gc, 1.0)
    neg = jnp.where(cm, out, -1e30)
    mx = jnp.max(neg, axis=1, keepdims=True)
    e = jnp.where(cm, jnp.exp(out - mx), 0.0)
    lse = jnp.log(jnp.sum(e, axis=1, keepdims=True))
    out_ref[...] = out - mx - lse


def _p4(s3, z3, batch2):
    return pl.pallas_call(
        _p4_body,
        grid=(1,),
        in_specs=[
            pl.BlockSpec((C_PAD, 16), lambda i: (0, 0)),
            pl.BlockSpec((C_PAD, 16), lambda i: (1, 0)),
            pl.BlockSpec((C_PAD, 16), lambda i: (0, 0)),
            pl.BlockSpec((C_PAD, 1), lambda i: (0, 0)),
        ],
        out_specs=pl.BlockSpec((BATCHES, 16), lambda i: (0, 0)),
        out_shape=jax.ShapeDtypeStruct((BATCHES, 16), jnp.float32),
    )(s3, s3, z3, batch2)


def kernel(x, edge_index, batch, W_in_l, W_in_r, b_in, W_h_l, W_h_r, b_h,
           W_out_l, W_out_r, b_out):
    pad = jnp.full((E_PAD - E,), P_IDX, jnp.int32)
    src = jnp.concatenate([edge_index[0], pad]).reshape(E_PAD // K, K)
    dst = jnp.concatenate([edge_index[1], pad]).reshape(E_PAD // K, K)
    x_pad = jnp.pad(x, ((0, N_PAD - N), (0, 0)))
    batch2 = jnp.concatenate(
        [batch[0::2], jnp.full((C_PAD - C_REAL,), BATCHES, jnp.int32)]
    ).reshape(C_PAD, 1)
    Wl3 = jnp.pad(W_out_l, ((0, 0), (0, 6)))
    Wr3 = jnp.pad(W_out_r, ((0, 0), (0, 6)))
    bo = jnp.pad(b_out, (0, 6)).reshape(1, 16)
    bi = b_in.reshape(1, 64)
    bh = b_h.reshape(1, 64)

    y1e, z1 = _p1(x_pad, W_in_l, W_in_r)
    s1 = _seg_sum_sc(y1e, src, dst, N_PAD, 80, False)
    y2, z2 = _p2(s1, z1, bi, W_h_l, W_h_r)
    s2 = _seg_sum_sc(y2, src, dst, N_PAD, 64, False)
    y3, z3 = _p3(s2, s1, z2, bh, Wl3, Wr3, bo)
    s3 = _seg_sum_sc(y3, src, dst, C_PAD, 16, True)
    out = _p4(s3, z3, batch2)
    return out[:, :10]


# speedup vs baseline: 15.6823x; 1.1309x over previous
import functools

import jax
import jax.numpy as jnp
from jax import lax
from jax.experimental import pallas as pl
from jax.experimental.pallas import tpu as pltpu
from jax.experimental.pallas import tpu_sc as plsc

N = 10000
E = 320000
BATCHES = 8
N_PAD = 10240
C_REAL = 5000
C_PAD = 5120
E_PAD = 327680
P_IDX = 10100
K = 128
NC = 2
NS = 16
RB = 1024


def _make_sc_seg_sum(n_rows, width, shift, ch_fast, ch_slow):
    rows_per_tile = n_rows // NS
    assert NS * (ch_fast + ch_slow) * K == E_PAD
    NB = 5
    ZR = 64
    mesh = plsc.VectorSubcoreMesh(core_axis_name="c", subcore_axis_name="s")

    @functools.partial(
        pl.kernel,
        out_type=jax.ShapeDtypeStruct((NC * n_rows, width), jnp.float32),
        mesh=mesh,
        scratch_types=[
            pltpu.VMEM((NB, K), jnp.int32),
            pltpu.VMEM((NB, K), jnp.int32),
            pltpu.VMEM((NB, K, width), jnp.float32),
            pltpu.VMEM((ZR, width), jnp.float32),
            pltpu.VMEM_SHARED((n_rows, width), jnp.float32),
            pltpu.SemaphoreType.DMA,
            pltpu.SemaphoreType.DMA,
        ],
        compiler_params=pltpu.CompilerParams(use_tc_tiling_on_sc=False),
    )
    def seg_sum(tab_hbm, src_hbm, dst_hbm, out_hbm, src_v, dst_v, rows_v,
                zb_v, acc_sh, gsem, ssem):
        c = lax.axis_index("c")
        s = lax.axis_index("s")
        zero16 = jnp.zeros((16,), jnp.float32)
        for r in range(ZR):
            for j in range(width // 16):
                zb_v[r, pl.ds(j * 16, 16)] = zero16
        r0 = s * rows_per_tile
        for t in range(rows_per_tile // ZR):
            pltpu.sync_copy(zb_v, acc_sh.at[pl.ds(r0 + t * ZR, ZR)])
        plsc.subcore_barrier()

        is_fast = c == 0
        base_chunk = jnp.where(is_fast, s * ch_fast,
                               NS * ch_fast + s * ch_slow)
        ngroups = jnp.where(is_fast, ch_fast // NB, ch_slow // NB)

        def body(t, carry):
            row0 = base_chunk + t * NB
            pltpu.sync_copy(src_hbm.at[pl.ds(row0, NB)], src_v)
            pltpu.sync_copy(dst_hbm.at[pl.ds(row0, NB)], dst_v)
            if shift:
                for b in range(NB):
                    for j in range(K // 16):
                        sl = pl.ds(j * 16, 16)
                        src_v[b, sl] = lax.shift_right_logical(src_v[b, sl], 1)
                        dst_v[b, sl] = lax.shift_right_logical(dst_v[b, sl], 1)
            gds = [pltpu.async_copy(tab_hbm.at[src_v.at[b]], rows_v.at[b], gsem)
                   for b in range(NB)]
            sds = []
            for b in range(NB):
                gds[b].wait()
                sds.append(pltpu.async_copy(rows_v.at[b], acc_sh.at[dst_v.at[b]],
                                            ssem, add=True))
            for b in range(NB):
                sds[b].wait()
            return carry

        lax.fori_loop(0, ngroups, body, 0)
        plsc.subcore_barrier()
        pltpu.sync_copy(
            acc_sh.at[pl.ds(r0, rows_per_tile)],
            out_hbm.at[pl.ds(c * n_rows + r0, rows_per_tile)])

    return seg_sum


@functools.cache
def _get_sc_seg_sum(n_rows, width, shift, ch_fast, ch_slow):
    return _make_sc_seg_sum(n_rows, width, shift, ch_fast, ch_slow)


def _seg_sum_sc(table, src, dst, n_rows, width, shift):
    ch_fast, ch_slow = (100, 60) if shift else (120, 40)
    return _get_sc_seg_sum(n_rows, width, shift, ch_fast, ch_slow)(
        table, src, dst)


def _p1_body(x_ref, wl_ref, wr_ref, y1e_ref, z1_ref):
    i = pl.program_id(0)
    xb = x_ref[...]
    y = jnp.dot(xb, wl_ref[...], preferred_element_type=jnp.float32)
    rows = i * RB + lax.broadcasted_iota(jnp.int32, (RB, 1), 0)
    rmask = jnp.where(rows < N, 1.0, 0.0)
    col16 = lax.broadcasted_iota(jnp.int32, (1, 16), 1)
    extra = jnp.where(col16 == 0, rmask, 0.0)
    y1e_ref[...] = jnp.concatenate([y, extra], axis=1)
    z1_ref[...] = jnp.dot(xb, wr_ref[...], preferred_element_type=jnp.float32)


def _p1(x_pad, W_in_l, W_in_r):
    return pl.pallas_call(
        _p1_body,
        grid=(N_PAD // RB,),
        in_specs=[
            pl.BlockSpec((RB, 128), lambda i: (i, 0)),
            pl.BlockSpec((128, 64), lambda i: (0, 0)),
            pl.BlockSpec((128, 64), lambda i: (0, 0)),
        ],
        out_specs=[
            pl.BlockSpec((RB, 80), lambda i: (i, 0)),
            pl.BlockSpec((RB, 64), lambda i: (i, 0)),
        ],
        out_shape=[
            jax.ShapeDtypeStruct((N_PAD, 80), jnp.float32),
            jax.ShapeDtypeStruct((N_PAD, 64), jnp.float32),
        ],
    )(x_pad, W_in_l, W_in_r)


def _p2_body(sa_ref, sb_ref, z1_ref, bi_ref, whl_ref, whr_ref, y2_ref, z2_ref):
    i = pl.program_id(0)
    sblk = sa_ref[...] + sb_ref[...]
    cnt = jnp.maximum(sblk[:, 64:65], 1.0)
    h = sblk[:, :64] / cnt + z1_ref[...] + bi_ref[...]
    h = jnp.maximum(h, 0.0)
    nrm = jnp.maximum(jnp.sqrt(jnp.sum(h * h, axis=1, keepdims=True)), 1e-12)
    h = h / nrm
    rows = i * RB + lax.broadcasted_iota(jnp.int32, (RB, 1), 0)
    h = jnp.where(rows < N, h, 0.0)
    y2_ref[...] = jnp.dot(h, whl_ref[...], preferred_element_type=jnp.float32)
    z2_ref[...] = jnp.dot(h, whr_ref[...], preferred_element_type=jnp.float32)


def _p2(s1, z1, bi, W_h_l, W_h_r):
    nb = N_PAD // RB
    return pl.pallas_call(
        _p2_body,
        grid=(nb,),
        in_specs=[
            pl.BlockSpec((RB, 80), lambda i: (i, 0)),
            pl.BlockSpec((RB, 80), lambda i, nb=nb: (nb + i, 0)),
            pl.BlockSpec((RB, 64), lambda i: (i, 0)),
            pl.BlockSpec((1, 64), lambda i: (0, 0)),
            pl.BlockSpec((64, 64), lambda i: (0, 0)),
            pl.BlockSpec((64, 64), lambda i: (0, 0)),
        ],
        out_specs=[
            pl.BlockSpec((RB, 64), lambda i: (i, 0)),
            pl.BlockSpec((RB, 64), lambda i: (i, 0)),
        ],
        out_shape=[
            jax.ShapeDtypeStruct((N_PAD, 64), jnp.float32),
            jax.ShapeDtypeStruct((N_PAD, 64), jnp.float32),
        ],
    )(s1, s1, z1, bi, W_h_l, W_h_r)


def _p3_body(sa_ref, sb_ref, ca_ref, cb_ref, z2_ref, bh_ref, wol_ref, wor_ref,
             bo_ref, y3_ref, z3_ref):
    i = pl.program_id(0)
    cnt = jnp.maximum(ca_ref[:, 64:65] + cb_ref[:, 64:65], 1.0)
    sblk = sa_ref[...] + sb_ref[...]
    h = jnp.maximum(sblk / cnt + z2_ref[...] + bh_ref[...], 0.0)
    nrm = jnp.maximum(jnp.sqrt(jnp.sum(h * h, axis=1, keepdims=True)), 1e-12)
    h = h / nrm
    rows = i * RB + lax.broadcasted_iota(jnp.int32, (RB, 1), 0)
    h = jnp.where(rows < N, h, 0.0)
    rj = lax.broadcasted_iota(jnp.int32, (RB // 2, RB), 0)
    ci = lax.broadcasted_iota(jnp.int32, (RB // 2, RB), 1)
    pair = jnp.where(lax.shift_right_logical(ci, 1) == rj, 0.5, 0.0)
    x2 = jnp.dot(pair, h, preferred_element_type=jnp.float32)
    gc = i * (RB // 2) + lax.broadcasted_iota(jnp.int32, (RB // 2, 1), 0)
    cmask = gc < C_REAL
    col16 = lax.broadcasted_iota(jnp.int32, (1, 16), 1)
    y3 = jnp.dot(x2, wol_ref[...], preferred_element_type=jnp.float32)
    y3 = y3 + jnp.where(col16 == 10, 1.0, 0.0)
    y3_ref[...] = jnp.where(cmask, y3, 0.0)
    z3_ref[...] = (jnp.dot(x2, wor_ref[...], preferred_element_type=jnp.float32)
                   + bo_ref[...])


def _p3(s2, s1, z2, bh, Wl3, Wr3, bo):
    nb = N_PAD // RB
    return pl.pallas_call(
        _p3_body,
        grid=(nb,),
        in_specs=[
            pl.BlockSpec((RB, 64), lambda i: (i, 0)),
            pl.BlockSpec((RB, 64), lambda i, nb=nb: (nb + i, 0)),
            pl.BlockSpec((RB, 80), lambda i: (i, 0)),
            pl.BlockSpec((RB, 80), lambda i, nb=nb: (nb + i, 0)),
            pl.BlockSpec((RB, 64), lambda i: (i, 0)),
            pl.BlockSpec((1, 64), lambda i: (0, 0)),
            pl.BlockSpec((64, 16), lambda i: (0, 0)),
            pl.BlockSpec((64, 16), lambda i: (0, 0)),
            pl.BlockSpec((1, 16), lambda i: (0, 0)),
        ],
        out_specs=[
            pl.BlockSpec((RB // 2, 16), lambda i: (i, 0)),
            pl.BlockSpec((RB // 2, 16), lambda i: (i, 0)),
        ],
        out_shape=[
            jax.ShapeDtypeStruct((C_PAD, 16), jnp.float32),
            jax.ShapeDtypeStruct((C_PAD, 16), jnp.float32),
        ],
    )(s2, s2, s1, s1, z2, bh, Wl3, Wr3, bo)


def _p4_body(sa_ref, sb_ref, z3_ref, b2_ref, out_ref):
    sblk = sa_ref[...] + sb_ref[...]
    cnt2 = jnp.maximum(sblk[:, 10:11], 1.0)
    o = sblk / cnt2 + z3_ref[...]
    col16 = lax.broadcasted_iota(jnp.int32, (1, 16), 1)
    cm = col16 < 10
    o = jnp.where(cm, o, 0.0)
    nrm = jnp.maximum(jnp.sqrt(jnp.sum(o * o, axis=1, keepdims=True)), 1e-12)
    o = o / nrm
    b2 = b2_ref[...]
    col8 = lax.broadcasted_iota(jnp.int32, (1, BATCHES), 1)
    onehot = jnp.where(b2 == col8, 1.0, 0.0)
    dn = (((0,), (0,)), ((), ()))
    gs = lax.dot_general(onehot, o, dn, preferred_element_type=jnp.float32)
    gc = lax.dot_general(onehot, jnp.ones_like(o), dn,
                         preferred_element_type=jnp.float32)
    out = gs / jnp.maximum(gc, 1.0)
    neg = jnp.where(cm, out, -1e30)
    mx = jnp.max(neg, axis=1, keepdims=True)
    e = jnp.where(cm, jnp.exp(out - mx), 0.0)
    lse = jnp.log(jnp.sum(e, axis=1, keepdims=True))
    out_ref[...] = out - mx - lse


def _p4(s3, z3, batch2):
    return pl.pallas_call(
        _p4_body,
        grid=(1,),
        in_specs=[
            pl.BlockSpec((C_PAD, 16), lambda i: (0, 0)),
            pl.BlockSpec((C_PAD, 16), lambda i: (1, 0)),
            pl.BlockSpec((C_PAD, 16), lambda i: (0, 0)),
            pl.BlockSpec((C_PAD, 1), lambda i: (0, 0)),
        ],
        out_specs=pl.BlockSpec((BATCHES, 16), lambda i: (0, 0)),
        out_shape=jax.ShapeDtypeStruct((BATCHES, 16), jnp.float32),
    )(s3, s3, z3, batch2)


def kernel(x, edge_index, batch, W_in_l, W_in_r, b_in, W_h_l, W_h_r, b_h,
           W_out_l, W_out_r, b_out):
    pad = jnp.full((E_PAD - E,), P_IDX, jnp.int32)
    src = jnp.concatenate([edge_index[0], pad]).reshape(E_PAD // K, K)
    dst = jnp.concatenate([edge_index[1], pad]).reshape(E_PAD // K, K)
    x_pad = jnp.pad(x, ((0, N_PAD - N), (0, 0)))
    batch2 = jnp.concatenate(
        [batch[0::2], jnp.full((C_PAD - C_REAL,), BATCHES, jnp.int32)]
    ).reshape(C_PAD, 1)
    Wl3 = jnp.pad(W_out_l, ((0, 0), (0, 6)))
    Wr3 = jnp.pad(W_out_r, ((0, 0), (0, 6)))
    bo = jnp.pad(b_out, (0, 6)).reshape(1, 16)
    bi = b_in.reshape(1, 64)
    bh = b_h.reshape(1, 64)

    y1e, z1 = _p1(x_pad, W_in_l, W_in_r)
    s1 = _seg_sum_sc(y1e, src, dst, N_PAD, 80, False)
    y2, z2 = _p2(s1, z1, bi, W_h_l, W_h_r)
    s2 = _seg_sum_sc(y2, src, dst, N_PAD, 64, False)
    y3, z3 = _p3(s2, s1, z2, bh, Wl3, Wr3, bo)
    s3 = _seg_sum_sc(y3, src, dst, C_PAD, 16, True)
    out = _p4(s3, z3, batch2)
    return out[:, :10]
